# Initial kernel scaffold; baseline (speedup 1.0000x reference)
import jax
import jax.numpy as jnp
from jax.experimental import pallas as pl


def kernel(x, o, W1, b1, W2, b2, Wm, bm, Ws, bs):
    raise NotImplementedError("write your pallas kernel here")


# speedup vs baseline: 1.4557x; 1.4557x over previous
import functools

import jax
import jax.numpy as jnp
from jax import lax
from jax.experimental import pallas as pl
from jax.experimental.pallas import tpu as pltpu
from jax.experimental.pallas import tpu_sc as plsc

LS_MAX = 2.0
LS_MIN = -5.0

NC, NS = 2, 16
NW = NC * NS
L = 16
BLK = 128
XCH = 32


def _wid():
    return lax.axis_index("s") * NC + lax.axis_index("c")


def _mesh():
    return plsc.VectorSubcoreMesh(
        core_axis_name="c", subcore_axis_name="s",
        num_cores=NC, num_subcores=NS)


def _sc_count(o, B, E):
    TPW = B // NW

    @functools.partial(
        pl.kernel, mesh=_mesh(),
        compiler_params=pltpu.CompilerParams(needs_layout_passes=False),
        out_type=(jax.ShapeDtypeStruct((NW, E), jnp.int32),
                  jax.ShapeDtypeStruct((NW, TPW), jnp.int32)),
        scratch_types=[
            pltpu.VMEM((TPW,), jnp.int32),
            pltpu.VMEM((TPW,), jnp.int32),
            pltpu.VMEM((E,), jnp.int32),
        ],
    )
    def k(o_hbm, cnt_hbm, lpos_hbm, o_v, lp_v, c_v):
        w = _wid()
        pltpu.sync_copy(o_hbm.at[pl.ds(w * TPW, TPW)], o_v)
        lanes = lax.iota(jnp.int32, L)

        def sub(kk, cnt):
            ov = o_v[pl.ds(kk * L, L)]
            lp = jnp.zeros((L,), jnp.int32)
            for e in range(E):
                m = ov == e
                m32 = m.astype(jnp.int32)
                csum = plsc.cumsum(m32)
                lp = jnp.where(m, cnt[e] + csum - 1, lp)
                pc = plsc.all_reduce_population_count(m)
                cnt = cnt + jnp.where(lanes == e, pc,
                                      jnp.zeros((L,), jnp.int32))
            lp_v[pl.ds(kk * L, L)] = lp
            return cnt

        cnt = lax.fori_loop(0, TPW // L, sub, jnp.zeros((E,), jnp.int32))
        c_v[...] = cnt
        pltpu.sync_copy(lp_v, lpos_hbm.at[w])
        pltpu.sync_copy(c_v, cnt_hbm.at[w])

    return k(o)


def _sc_dispatch(o, counts, lpos, x, B, D, E, NBMAX):
    TPW = B // NW
    NCH = TPW // XCH
    B_pad = NBMAX * BLK

    @functools.partial(
        pl.kernel, mesh=_mesh(),
        compiler_params=pltpu.CompilerParams(needs_layout_passes=False),
        out_type=(jax.ShapeDtypeStruct((B_pad, D), jnp.float32),
                  jax.ShapeDtypeStruct((NW, NCH, XCH), jnp.int32),
                  jax.ShapeDtypeStruct((NBMAX,), jnp.int32),
                  jax.ShapeDtypeStruct((L,), jnp.int32)),
        scratch_types=[
            pltpu.VMEM((NW, E), jnp.int32),
            pltpu.VMEM((NCH, XCH), jnp.int32),
            pltpu.VMEM((NBMAX,), jnp.int32),
            pltpu.VMEM((L,), jnp.int32),
            pltpu.VMEM((XCH, D), jnp.float32),
            pltpu.VMEM((XCH, D), jnp.float32),
            pltpu.VMEM((TPW,), jnp.int32),
            pltpu.VMEM((TPW,), jnp.int32),
            pltpu.VMEM((E,), jnp.int32),
            pltpu.SemaphoreType.DMA,
            pltpu.SemaphoreType.DMA,
        ],
    )
    def k(o_hbm, cnt_hbm, lpos_hbm, x_hbm,
          xs_hbm, tts_hbm, be_hbm, nb_hbm,
          cv, ts_v, be_v, nb_v, xb0, xb1, o_v, lp_v, off_v, sem0, sem1):
        w = _wid()
        pltpu.sync_copy(cnt_hbm, cv)
        pltpu.sync_copy(o_hbm.at[pl.ds(w * TPW, TPW)], o_v)
        pltpu.sync_copy(lpos_hbm.at[w], lp_v)

        def tsum(t, acc):
            return acc + cv[t]
        tot = lax.fori_loop(0, NW, tsum, jnp.zeros((E,), jnp.int32))
        nblk = (tot + (BLK - 1)) // BLK
        sizes = nblk * BLK
        base = plsc.cumsum(sizes) - sizes

        def osum(t, acc):
            return acc + cv[t]
        off = base + lax.fori_loop(0, w, osum, jnp.zeros((E,), jnp.int32))
        off_v[...] = off

        for kk in range(TPW // L):
            ov = o_v[pl.ds(kk * L, L)]
            lp = lp_v[pl.ds(kk * L, L)]
            ts = plsc.load_gather(off_v, [ov]) + lp
            ts_v[kk // 2, pl.ds((kk % 2) * L, L)] = ts
        pltpu.sync_copy(ts_v, tts_hbm.at[w])

        @pl.when(w == 0)
        def _():
            nb_used = jnp.sum(nblk)
            for g in range(NBMAX // L):
                starts = (lax.iota(jnp.int32, L) + g * L) * BLK
                acc = jnp.zeros((L,), jnp.int32)
                for e in range(E):
                    acc = acc + (base[e] <= starts).astype(jnp.int32)
                be_v[pl.ds(g * L, L)] = acc - 1
            nb_v[...] = jnp.full((L,), nb_used, jnp.int32)
            pltpu.sync_copy(be_v, be_hbm)
            pltpu.sync_copy(nb_v, nb_hbm)

        pending = [None, None]
        for j in range(NCH):
            xb = xb0 if j % 2 == 0 else xb1
            sem = sem0 if j % 2 == 0 else sem1
            if pending[j % 2] is not None:
                pending[j % 2].wait()
            pltpu.sync_copy(x_hbm.at[pl.ds(w * TPW + j * XCH, XCH)], xb)
            cp = pltpu.make_async_copy(xb, xs_hbm.at[ts_v.at[j]], sem)
            cp.start()
            pending[j % 2] = cp
        pending[0].wait()
        pending[1].wait()

    return k(o, counts, lpos, x)


def _tc_mlp(be, nbv, xs, W1, b1r, W2, b2r, Wm, bmr, Ws, bsr,
            NBMAX, D, H, A):
    def body(be_ref, nb_ref, xs_ref, W1_ref, b1_ref, W2_ref, b2_ref,
             Wm_ref, bm_ref, Ws_ref, bs_ref, out_ref):
        i = pl.program_id(0)

        @pl.when(i < nb_ref[0])
        def _():
            hi = lax.Precision.HIGHEST
            xb = xs_ref[...]
            h1 = jnp.maximum(
                jnp.dot(xb, W1_ref[0], precision=hi,
                        preferred_element_type=jnp.float32) + b1_ref[0, 0], 0.0)
            h2 = jnp.maximum(
                jnp.dot(h1, W2_ref[0], precision=hi,
                        preferred_element_type=jnp.float32) + b2_ref[0, 0], 0.0)
            mean = jnp.dot(h2, Wm_ref[0], precision=hi,
                           preferred_element_type=jnp.float32) + bm_ref[0, 0]
            ls = jnp.dot(h2, Ws_ref[0], precision=hi,
                         preferred_element_type=jnp.float32) + bs_ref[0, 0]
            ls = LS_MIN + 0.5 * (LS_MAX - LS_MIN) * (jnp.tanh(ls) + 1.0)
            out_ref[...] = jnp.concatenate([mean, ls], axis=1)

    def xmap(i, be_r, nb_r):
        return (jnp.where(i < nb_r[0], i, nb_r[0] - 1), 0)

    def emap(i, be_r, nb_r):
        return (be_r[i], 0, 0)

    grid_spec = pltpu.PrefetchScalarGridSpec(
        num_scalar_prefetch=2,
        grid=(NBMAX,),
        in_specs=[
            pl.BlockSpec((BLK, D), xmap),
            pl.BlockSpec((1, D, H), emap),
            pl.BlockSpec((1, 1, H), emap),
            pl.BlockSpec((1, H, H), emap),
            pl.BlockSpec((1, 1, H), emap),
            pl.BlockSpec((1, H, A), emap),
            pl.BlockSpec((1, 1, A), emap),
            pl.BlockSpec((1, H, A), emap),
            pl.BlockSpec((1, 1, A), emap),
        ],
        out_specs=pl.BlockSpec((BLK, 2 * A), lambda i, be_r, nb_r: (i, 0)),
    )
    return pl.pallas_call(
        body,
        grid_spec=grid_spec,
        out_shape=jax.ShapeDtypeStruct((NBMAX * BLK, 2 * A), jnp.float32),
    )(be, nbv, xs, W1, b1r, W2, b2r, Wm, bmr, Ws, bsr)


def _sc_unsort(tts, comb, B, W2A):
    TPW = B // NW
    NCH = TPW // XCH

    @functools.partial(
        pl.kernel, mesh=_mesh(),
        compiler_params=pltpu.CompilerParams(needs_layout_passes=False),
        out_type=jax.ShapeDtypeStruct((B, W2A), jnp.float32),
        scratch_types=[
            pltpu.VMEM((NCH, XCH), jnp.int32),
            pltpu.VMEM((XCH, W2A), jnp.float32),
            pltpu.VMEM((XCH, W2A), jnp.float32),
            pltpu.SemaphoreType.DMA,
            pltpu.SemaphoreType.DMA,
        ],
    )
    def k(tts_hbm, comb_hbm, out_hbm, ts_v, g0, g1, sem0, sem1):
        w = _wid()
        pltpu.sync_copy(tts_hbm.at[w], ts_v)
        bufs = (g0, g1)
        sems = (sem0, sem1)
        pending = [None, None]
        pending[0] = pltpu.make_async_copy(comb_hbm.at[ts_v.at[0]],
                                           bufs[0], sems[0])
        pending[0].start()
        for j in range(NCH):
            if j + 1 < NCH:
                nxt = pltpu.make_async_copy(comb_hbm.at[ts_v.at[j + 1]],
                                            bufs[(j + 1) % 2], sems[(j + 1) % 2])
                nxt.start()
                pending[(j + 1) % 2] = nxt
            pending[j % 2].wait()
            pltpu.sync_copy(bufs[j % 2],
                            out_hbm.at[pl.ds(w * TPW + j * XCH, XCH)])

    return k(tts, comb)


def kernel(x, o, W1, b1, W2, b2, Wm, bm, Ws, bs):
    B, D = x.shape
    E, _, H = W1.shape
    A = Wm.shape[2]
    NBMAX = B // BLK + E

    o32 = o.astype(jnp.int32)
    counts, lpos = _sc_count(o32, B, E)
    xs, tts, be, nbv = _sc_dispatch(o32, counts, lpos, x, B, D, E, NBMAX)
    comb = _tc_mlp(be, nbv, xs, W1, b1[:, None, :], W2, b2[:, None, :],
                   Wm, bm[:, None, :], Ws, bs[:, None, :], NBMAX, D, H, A)
    out = _sc_unsort(tts, comb, B, 2 * A)
    return out[:, :A], out[:, A:]


# speedup vs baseline: 2.1435x; 1.4725x over previous
import functools

import jax
import jax.numpy as jnp
from jax import lax
from jax.experimental import pallas as pl
from jax.experimental.pallas import tpu as pltpu
from jax.experimental.pallas import tpu_sc as plsc

LS_MAX = 2.0
LS_MIN = -5.0

NC, NS = 2, 16
NW = NC * NS
L = 16
BLK = 128
XCH = 32


def _wid():
    return lax.axis_index("s") * NC + lax.axis_index("c")


def _mesh():
    return plsc.VectorSubcoreMesh(
        core_axis_name="c", subcore_axis_name="s",
        num_cores=NC, num_subcores=NS)


def _sc_count(o, B, E):
    TPW = B // NW

    @functools.partial(
        pl.kernel, mesh=_mesh(),
        compiler_params=pltpu.CompilerParams(needs_layout_passes=False),
        out_type=(jax.ShapeDtypeStruct((NW, E), jnp.int32),
                  jax.ShapeDtypeStruct((NW, TPW), jnp.int32)),
        scratch_types=[
            pltpu.VMEM((TPW,), jnp.int32),
            pltpu.VMEM((TPW,), jnp.int32),
            pltpu.VMEM((E,), jnp.int32),
        ],
    )
    def k(o_hbm, cnt_hbm, lpos_hbm, o_v, lp_v, c_v):
        w = _wid()
        pltpu.sync_copy(o_hbm.at[pl.ds(w * TPW, TPW)], o_v)
        lanes = lax.iota(jnp.int32, L)

        def sub(kk, cnt):
            ov = o_v[pl.ds(kk * L, L)]
            lp = jnp.zeros((L,), jnp.int32)
            for e in range(E):
                m = ov == e
                m32 = m.astype(jnp.int32)
                csum = plsc.cumsum(m32)
                lp = jnp.where(m, cnt[e] + csum - 1, lp)
                pc = plsc.all_reduce_population_count(m)
                cnt = cnt + jnp.where(lanes == e, pc,
                                      jnp.zeros((L,), jnp.int32))
            lp_v[pl.ds(kk * L, L)] = lp
            return cnt

        cnt = lax.fori_loop(0, TPW // L, sub, jnp.zeros((E,), jnp.int32))
        c_v[...] = cnt
        pltpu.sync_copy(lp_v, lpos_hbm.at[w])
        pltpu.sync_copy(c_v, cnt_hbm.at[w])

    return k(o)


def _sc_dispatch(o, counts, lpos, x, B, D, E, NBMAX):
    TPW = B // NW
    NCH = TPW // XCH
    B_pad = NBMAX * BLK

    @functools.partial(
        pl.kernel, mesh=_mesh(),
        compiler_params=pltpu.CompilerParams(needs_layout_passes=False),
        out_type=(jax.ShapeDtypeStruct((B_pad, D), jnp.float32),
                  jax.ShapeDtypeStruct((NW, NCH, XCH), jnp.int32),
                  jax.ShapeDtypeStruct((NBMAX,), jnp.int32),
                  jax.ShapeDtypeStruct((L,), jnp.int32)),
        scratch_types=[
            pltpu.VMEM((NW, E), jnp.int32),
            pltpu.VMEM((NCH, XCH), jnp.int32),
            pltpu.VMEM((NBMAX,), jnp.int32),
            pltpu.VMEM((L,), jnp.int32),
            pltpu.VMEM((XCH, D), jnp.float32),
            pltpu.VMEM((XCH, D), jnp.float32),
            pltpu.VMEM((TPW,), jnp.int32),
            pltpu.VMEM((TPW,), jnp.int32),
            pltpu.VMEM((E,), jnp.int32),
            pltpu.SemaphoreType.DMA,
            pltpu.SemaphoreType.DMA,
        ],
    )
    def k(o_hbm, cnt_hbm, lpos_hbm, x_hbm,
          xs_hbm, tts_hbm, be_hbm, nb_hbm,
          cv, ts_v, be_v, nb_v, xb0, xb1, o_v, lp_v, off_v, sem0, sem1):
        w = _wid()
        pltpu.sync_copy(cnt_hbm, cv)
        pltpu.sync_copy(o_hbm.at[pl.ds(w * TPW, TPW)], o_v)
        pltpu.sync_copy(lpos_hbm.at[w], lp_v)

        def tsum(t, acc):
            return acc + cv[t]
        tot = lax.fori_loop(0, NW, tsum, jnp.zeros((E,), jnp.int32))
        nblk = (tot + (BLK - 1)) // BLK
        sizes = nblk * BLK
        base = plsc.cumsum(sizes) - sizes

        def osum(t, acc):
            return acc + cv[t]
        off = base + lax.fori_loop(0, w, osum, jnp.zeros((E,), jnp.int32))
        off_v[...] = off

        for kk in range(TPW // L):
            ov = o_v[pl.ds(kk * L, L)]
            lp = lp_v[pl.ds(kk * L, L)]
            ts = plsc.load_gather(off_v, [ov]) + lp
            ts_v[kk // 2, pl.ds((kk % 2) * L, L)] = ts
        pltpu.sync_copy(ts_v, tts_hbm.at[w])

        @pl.when(w == 0)
        def _():
            nb_used = jnp.sum(nblk)
            for g in range(NBMAX // L):
                starts = (lax.iota(jnp.int32, L) + g * L) * BLK
                acc = jnp.zeros((L,), jnp.int32)
                for e in range(E):
                    acc = acc + (base[e] <= starts).astype(jnp.int32)
                be_v[pl.ds(g * L, L)] = acc - 1
            nb_v[...] = jnp.full((L,), nb_used, jnp.int32)
            pltpu.sync_copy(be_v, be_hbm)
            pltpu.sync_copy(nb_v, nb_hbm)

        pending = [None, None]
        for j in range(NCH):
            xb = xb0 if j % 2 == 0 else xb1
            sem = sem0 if j % 2 == 0 else sem1
            if pending[j % 2] is not None:
                pending[j % 2].wait()
            pltpu.sync_copy(x_hbm.at[pl.ds(w * TPW + j * XCH, XCH)], xb)
            cp = pltpu.make_async_copy(xb, xs_hbm.at[ts_v.at[j]], sem)
            cp.start()
            pending[j % 2] = cp
        pending[0].wait()
        pending[1].wait()

    return k(o, counts, lpos, x)


def _tc_mlp(be, nbv, xs, W1, b1r, W2, b2r, Wm, bmr, Ws, bsr,
            NBMAX, D, H, A):
    def body(be_ref, nb_ref, xs_ref, W1_ref, b1_ref, W2_ref, b2_ref,
             Wm_ref, bm_ref, Ws_ref, bs_ref, out_ref):
        i = pl.program_id(0)

        @pl.when(i < nb_ref[0])
        def _():
            hi = lax.Precision.DEFAULT
            xb = xs_ref[...]
            h1 = jnp.maximum(
                jnp.dot(xb, W1_ref[0], precision=hi,
                        preferred_element_type=jnp.float32) + b1_ref[0, 0], 0.0)
            h2 = jnp.maximum(
                jnp.dot(h1, W2_ref[0], precision=hi,
                        preferred_element_type=jnp.float32) + b2_ref[0, 0], 0.0)
            mean = jnp.dot(h2, Wm_ref[0], precision=hi,
                           preferred_element_type=jnp.float32) + bm_ref[0, 0]
            ls = jnp.dot(h2, Ws_ref[0], precision=hi,
                         preferred_element_type=jnp.float32) + bs_ref[0, 0]
            ls = LS_MIN + 0.5 * (LS_MAX - LS_MIN) * (jnp.tanh(ls) + 1.0)
            out_ref[...] = jnp.concatenate([mean, ls], axis=1)

    def xmap(i, be_r, nb_r):
        return (jnp.where(i < nb_r[0], i, nb_r[0] - 1), 0)

    def emap(i, be_r, nb_r):
        return (be_r[i], 0, 0)

    grid_spec = pltpu.PrefetchScalarGridSpec(
        num_scalar_prefetch=2,
        grid=(NBMAX,),
        in_specs=[
            pl.BlockSpec((BLK, D), xmap),
            pl.BlockSpec((1, D, H), emap),
            pl.BlockSpec((1, 1, H), emap),
            pl.BlockSpec((1, H, H), emap),
            pl.BlockSpec((1, 1, H), emap),
            pl.BlockSpec((1, H, A), emap),
            pl.BlockSpec((1, 1, A), emap),
            pl.BlockSpec((1, H, A), emap),
            pl.BlockSpec((1, 1, A), emap),
        ],
        out_specs=pl.BlockSpec((BLK, 2 * A), lambda i, be_r, nb_r: (i, 0)),
    )
    return pl.pallas_call(
        body,
        grid_spec=grid_spec,
        out_shape=jax.ShapeDtypeStruct((NBMAX * BLK, 2 * A), jnp.float32),
    )(be, nbv, xs, W1, b1r, W2, b2r, Wm, bmr, Ws, bsr)


def _sc_unsort(tts, comb, B, W2A):
    TPW = B // NW
    NCH = TPW // XCH

    @functools.partial(
        pl.kernel, mesh=_mesh(),
        compiler_params=pltpu.CompilerParams(needs_layout_passes=False),
        out_type=jax.ShapeDtypeStruct((B, W2A), jnp.float32),
        scratch_types=[
            pltpu.VMEM((NCH, XCH), jnp.int32),
            pltpu.VMEM((XCH, W2A), jnp.float32),
            pltpu.VMEM((XCH, W2A), jnp.float32),
            pltpu.SemaphoreType.DMA,
            pltpu.SemaphoreType.DMA,
        ],
    )
    def k(tts_hbm, comb_hbm, out_hbm, ts_v, g0, g1, sem0, sem1):
        w = _wid()
        pltpu.sync_copy(tts_hbm.at[w], ts_v)
        bufs = (g0, g1)
        sems = (sem0, sem1)
        pending = [None, None]
        pending[0] = pltpu.make_async_copy(comb_hbm.at[ts_v.at[0]],
                                           bufs[0], sems[0])
        pending[0].start()
        for j in range(NCH):
            if j + 1 < NCH:
                nxt = pltpu.make_async_copy(comb_hbm.at[ts_v.at[j + 1]],
                                            bufs[(j + 1) % 2], sems[(j + 1) % 2])
                nxt.start()
                pending[(j + 1) % 2] = nxt
            pending[j % 2].wait()
            pltpu.sync_copy(bufs[j % 2],
                            out_hbm.at[pl.ds(w * TPW + j * XCH, XCH)])

    return k(tts, comb)


def kernel(x, o, W1, b1, W2, b2, Wm, bm, Ws, bs):
    B, D = x.shape
    E, _, H = W1.shape
    A = Wm.shape[2]
    NBMAX = B // BLK + E

    o32 = o.astype(jnp.int32)
    counts, lpos = _sc_count(o32, B, E)
    xs, tts, be, nbv = _sc_dispatch(o32, counts, lpos, x, B, D, E, NBMAX)
    comb = _tc_mlp(be, nbv, xs, W1, b1[:, None, :], W2, b2[:, None, :],
                   Wm, bm[:, None, :], Ws, bs[:, None, :], NBMAX, D, H, A)
    out = _sc_unsort(tts, comb, B, 2 * A)
    return out[:, :A], out[:, A:]


# speedup vs baseline: 2.5255x; 1.1782x over previous
import functools

import jax
import jax.numpy as jnp
from jax import lax
from jax.experimental import pallas as pl
from jax.experimental.pallas import tpu as pltpu
from jax.experimental.pallas import tpu_sc as plsc

LS_MAX = 2.0
LS_MIN = -5.0

NC, NS = 2, 16
NW = NC * NS
L = 16
BLK = 256
XCH = 32


def _wid():
    return lax.axis_index("s") * NC + lax.axis_index("c")


def _mesh():
    return plsc.VectorSubcoreMesh(
        core_axis_name="c", subcore_axis_name="s",
        num_cores=NC, num_subcores=NS)


def _sc_count(o, B, E):
    TPW = B // NW

    @functools.partial(
        pl.kernel, mesh=_mesh(),
        compiler_params=pltpu.CompilerParams(needs_layout_passes=False),
        out_type=(jax.ShapeDtypeStruct((NW, E), jnp.int32),
                  jax.ShapeDtypeStruct((NW, TPW), jnp.int32)),
        scratch_types=[
            pltpu.VMEM((TPW,), jnp.int32),
            pltpu.VMEM((TPW,), jnp.int32),
            pltpu.VMEM((E,), jnp.int32),
        ],
    )
    def k(o_hbm, cnt_hbm, lpos_hbm, o_v, lp_v, c_v):
        w = _wid()
        pltpu.sync_copy(o_hbm.at[pl.ds(w * TPW, TPW)], o_v)
        lanes = lax.iota(jnp.int32, L)

        def sub(kk, cnt):
            ov = o_v[pl.ds(kk * L, L)]
            lp = jnp.zeros((L,), jnp.int32)
            for e in range(E):
                m = ov == e
                m32 = m.astype(jnp.int32)
                csum = plsc.cumsum(m32)
                lp = jnp.where(m, cnt[e] + csum - 1, lp)
                pc = plsc.all_reduce_population_count(m)
                cnt = cnt + jnp.where(lanes == e, pc,
                                      jnp.zeros((L,), jnp.int32))
            lp_v[pl.ds(kk * L, L)] = lp
            return cnt

        cnt = lax.fori_loop(0, TPW // L, sub, jnp.zeros((E,), jnp.int32))
        c_v[...] = cnt
        pltpu.sync_copy(lp_v, lpos_hbm.at[w])
        pltpu.sync_copy(c_v, cnt_hbm.at[w])

    return k(o)


def _sc_dispatch(o, counts, lpos, x, B, D, E, NBMAX):
    TPW = B // NW
    NCH = TPW // XCH
    B_pad = NBMAX * BLK

    @functools.partial(
        pl.kernel, mesh=_mesh(),
        compiler_params=pltpu.CompilerParams(needs_layout_passes=False),
        out_type=(jax.ShapeDtypeStruct((B_pad, D), jnp.float32),
                  jax.ShapeDtypeStruct((NW, NCH, XCH), jnp.int32),
                  jax.ShapeDtypeStruct((NBMAX,), jnp.int32),
                  jax.ShapeDtypeStruct((L,), jnp.int32)),
        scratch_types=[
            pltpu.VMEM((NW, E), jnp.int32),
            pltpu.VMEM((NCH, XCH), jnp.int32),
            pltpu.VMEM((NBMAX,), jnp.int32),
            pltpu.VMEM((L,), jnp.int32),
            pltpu.VMEM((XCH, D), jnp.float32),
            pltpu.VMEM((XCH, D), jnp.float32),
            pltpu.VMEM((TPW,), jnp.int32),
            pltpu.VMEM((TPW,), jnp.int32),
            pltpu.VMEM((E,), jnp.int32),
            pltpu.SemaphoreType.DMA,
            pltpu.SemaphoreType.DMA,
        ],
    )
    def k(o_hbm, cnt_hbm, lpos_hbm, x_hbm,
          xs_hbm, tts_hbm, be_hbm, nb_hbm,
          cv, ts_v, be_v, nb_v, xb0, xb1, o_v, lp_v, off_v, sem0, sem1):
        w = _wid()
        pltpu.sync_copy(cnt_hbm, cv)
        pltpu.sync_copy(o_hbm.at[pl.ds(w * TPW, TPW)], o_v)
        pltpu.sync_copy(lpos_hbm.at[w], lp_v)

        def tsum(t, acc):
            return acc + cv[t]
        tot = lax.fori_loop(0, NW, tsum, jnp.zeros((E,), jnp.int32))
        nblk = (tot + (BLK - 1)) // BLK
        sizes = nblk * BLK
        base = plsc.cumsum(sizes) - sizes

        def osum(t, acc):
            return acc + cv[t]
        off = base + lax.fori_loop(0, w, osum, jnp.zeros((E,), jnp.int32))
        off_v[...] = off

        for kk in range(TPW // L):
            ov = o_v[pl.ds(kk * L, L)]
            lp = lp_v[pl.ds(kk * L, L)]
            ts = plsc.load_gather(off_v, [ov]) + lp
            ts_v[kk // 2, pl.ds((kk % 2) * L, L)] = ts
        pltpu.sync_copy(ts_v, tts_hbm.at[w])

        @pl.when(w == 0)
        def _():
            nb_used = jnp.sum(nblk)
            for g in range(NBMAX // L):
                starts = (lax.iota(jnp.int32, L) + g * L) * BLK
                acc = jnp.zeros((L,), jnp.int32)
                for e in range(E):
                    acc = acc + (base[e] <= starts).astype(jnp.int32)
                be_v[pl.ds(g * L, L)] = acc - 1
            nb_v[...] = jnp.full((L,), nb_used, jnp.int32)
            pltpu.sync_copy(be_v, be_hbm)
            pltpu.sync_copy(nb_v, nb_hbm)

        pending = [None, None]
        for j in range(NCH):
            xb = xb0 if j % 2 == 0 else xb1
            sem = sem0 if j % 2 == 0 else sem1
            if pending[j % 2] is not None:
                pending[j % 2].wait()
            pltpu.sync_copy(x_hbm.at[pl.ds(w * TPW + j * XCH, XCH)], xb)
            cp = pltpu.make_async_copy(xb, xs_hbm.at[ts_v.at[j]], sem)
            cp.start()
            pending[j % 2] = cp
        pending[0].wait()
        pending[1].wait()

    return k(o, counts, lpos, x)


def _tc_mlp(be, nbv, xs, W1, b1r, W2, b2r, Wm, bmr, Ws, bsr,
            NBMAX, D, H, A):
    def body(be_ref, nb_ref, xs_ref, W1_ref, b1_ref, W2_ref, b2_ref,
             Wm_ref, bm_ref, Ws_ref, bs_ref, out_ref):
        i = pl.program_id(0)

        @pl.when(i < nb_ref[0])
        def _():
            hi = lax.Precision.DEFAULT
            xb = xs_ref[...]
            h1 = jnp.maximum(
                jnp.dot(xb, W1_ref[0], precision=hi,
                        preferred_element_type=jnp.float32) + b1_ref[0, 0], 0.0)
            h2 = jnp.maximum(
                jnp.dot(h1, W2_ref[0], precision=hi,
                        preferred_element_type=jnp.float32) + b2_ref[0, 0], 0.0)
            mean = jnp.dot(h2, Wm_ref[0], precision=hi,
                           preferred_element_type=jnp.float32) + bm_ref[0, 0]
            ls = jnp.dot(h2, Ws_ref[0], precision=hi,
                         preferred_element_type=jnp.float32) + bs_ref[0, 0]
            ls = LS_MIN + 0.5 * (LS_MAX - LS_MIN) * (jnp.tanh(ls) + 1.0)
            out_ref[...] = jnp.concatenate([mean, ls], axis=1)

    def xmap(i, be_r, nb_r):
        return (jnp.where(i < nb_r[0], i, nb_r[0] - 1), 0)

    def emap(i, be_r, nb_r):
        return (be_r[i], 0, 0)

    grid_spec = pltpu.PrefetchScalarGridSpec(
        num_scalar_prefetch=2,
        grid=(NBMAX,),
        in_specs=[
            pl.BlockSpec((BLK, D), xmap),
            pl.BlockSpec((1, D, H), emap),
            pl.BlockSpec((1, 1, H), emap),
            pl.BlockSpec((1, H, H), emap),
            pl.BlockSpec((1, 1, H), emap),
            pl.BlockSpec((1, H, A), emap),
            pl.BlockSpec((1, 1, A), emap),
            pl.BlockSpec((1, H, A), emap),
            pl.BlockSpec((1, 1, A), emap),
        ],
        out_specs=pl.BlockSpec((BLK, 2 * A), lambda i, be_r, nb_r: (i, 0)),
    )
    return pl.pallas_call(
        body,
        grid_spec=grid_spec,
        out_shape=jax.ShapeDtypeStruct((NBMAX * BLK, 2 * A), jnp.float32),
    )(be, nbv, xs, W1, b1r, W2, b2r, Wm, bmr, Ws, bsr)


def _sc_unsort(tts, comb, B, W2A):
    TPW = B // NW
    NCH = TPW // XCH

    @functools.partial(
        pl.kernel, mesh=_mesh(),
        compiler_params=pltpu.CompilerParams(needs_layout_passes=False),
        out_type=jax.ShapeDtypeStruct((B, W2A), jnp.float32),
        scratch_types=[
            pltpu.VMEM((NCH, XCH), jnp.int32),
            pltpu.VMEM((XCH, W2A), jnp.float32),
            pltpu.VMEM((XCH, W2A), jnp.float32),
            pltpu.SemaphoreType.DMA,
            pltpu.SemaphoreType.DMA,
        ],
    )
    def k(tts_hbm, comb_hbm, out_hbm, ts_v, g0, g1, sem0, sem1):
        w = _wid()
        pltpu.sync_copy(tts_hbm.at[w], ts_v)
        bufs = (g0, g1)
        sems = (sem0, sem1)
        pending = [None, None]
        pending[0] = pltpu.make_async_copy(comb_hbm.at[ts_v.at[0]],
                                           bufs[0], sems[0])
        pending[0].start()
        for j in range(NCH):
            if j + 1 < NCH:
                nxt = pltpu.make_async_copy(comb_hbm.at[ts_v.at[j + 1]],
                                            bufs[(j + 1) % 2], sems[(j + 1) % 2])
                nxt.start()
                pending[(j + 1) % 2] = nxt
            pending[j % 2].wait()
            pltpu.sync_copy(bufs[j % 2],
                            out_hbm.at[pl.ds(w * TPW + j * XCH, XCH)])

    return k(tts, comb)


def kernel(x, o, W1, b1, W2, b2, Wm, bm, Ws, bs):
    B, D = x.shape
    E, _, H = W1.shape
    A = Wm.shape[2]
    NBMAX = B // BLK + E

    o32 = o.astype(jnp.int32)
    counts, lpos = _sc_count(o32, B, E)
    xs, tts, be, nbv = _sc_dispatch(o32, counts, lpos, x, B, D, E, NBMAX)
    comb = _tc_mlp(be, nbv, xs, W1, b1[:, None, :], W2, b2[:, None, :],
                   Wm, bm[:, None, :], Ws, bs[:, None, :], NBMAX, D, H, A)
    out = _sc_unsort(tts, comb, B, 2 * A)
    return out[:, :A], out[:, A:]


# speedup vs baseline: 2.7041x; 1.0708x over previous
import functools

import jax
import jax.numpy as jnp
from jax import lax
from jax.experimental import pallas as pl
from jax.experimental.pallas import tpu as pltpu
from jax.experimental.pallas import tpu_sc as plsc

LS_MAX = 2.0
LS_MIN = -5.0

NC, NS = 2, 16
NW = NC * NS
L = 16
BLK = 512
XCH = 32


def _wid():
    return lax.axis_index("s") * NC + lax.axis_index("c")


def _mesh():
    return plsc.VectorSubcoreMesh(
        core_axis_name="c", subcore_axis_name="s",
        num_cores=NC, num_subcores=NS)


def _sc_count(o, B, E):
    TPW = B // NW

    @functools.partial(
        pl.kernel, mesh=_mesh(),
        compiler_params=pltpu.CompilerParams(needs_layout_passes=False),
        out_type=(jax.ShapeDtypeStruct((NW, E), jnp.int32),
                  jax.ShapeDtypeStruct((NW, TPW), jnp.int32)),
        scratch_types=[
            pltpu.VMEM((TPW,), jnp.int32),
            pltpu.VMEM((TPW,), jnp.int32),
            pltpu.VMEM((E,), jnp.int32),
        ],
    )
    def k(o_hbm, cnt_hbm, lpos_hbm, o_v, lp_v, c_v):
        w = _wid()
        pltpu.sync_copy(o_hbm.at[pl.ds(w * TPW, TPW)], o_v)
        lanes = lax.iota(jnp.int32, L)

        def sub(kk, cnt):
            ov = o_v[pl.ds(kk * L, L)]
            lp = jnp.zeros((L,), jnp.int32)
            for e in range(E):
                m = ov == e
                m32 = m.astype(jnp.int32)
                csum = plsc.cumsum(m32)
                lp = jnp.where(m, cnt[e] + csum - 1, lp)
                pc = plsc.all_reduce_population_count(m)
                cnt = cnt + jnp.where(lanes == e, pc,
                                      jnp.zeros((L,), jnp.int32))
            lp_v[pl.ds(kk * L, L)] = lp
            return cnt

        cnt = lax.fori_loop(0, TPW // L, sub, jnp.zeros((E,), jnp.int32))
        c_v[...] = cnt
        pltpu.sync_copy(lp_v, lpos_hbm.at[w])
        pltpu.sync_copy(c_v, cnt_hbm.at[w])

    return k(o)


def _sc_dispatch(o, counts, lpos, x, B, D, E, NBMAX):
    TPW = B // NW
    NCH = TPW // XCH
    B_pad = NBMAX * BLK

    @functools.partial(
        pl.kernel, mesh=_mesh(),
        compiler_params=pltpu.CompilerParams(needs_layout_passes=False),
        out_type=(jax.ShapeDtypeStruct((B_pad, D), jnp.float32),
                  jax.ShapeDtypeStruct((NW, NCH, XCH), jnp.int32),
                  jax.ShapeDtypeStruct((NBMAX,), jnp.int32),
                  jax.ShapeDtypeStruct((L,), jnp.int32)),
        scratch_types=[
            pltpu.VMEM((NW, E), jnp.int32),
            pltpu.VMEM((NCH, XCH), jnp.int32),
            pltpu.VMEM((NBMAX,), jnp.int32),
            pltpu.VMEM((L,), jnp.int32),
            pltpu.VMEM((XCH, D), jnp.float32),
            pltpu.VMEM((XCH, D), jnp.float32),
            pltpu.VMEM((TPW,), jnp.int32),
            pltpu.VMEM((TPW,), jnp.int32),
            pltpu.VMEM((E,), jnp.int32),
            pltpu.SemaphoreType.DMA,
            pltpu.SemaphoreType.DMA,
        ],
    )
    def k(o_hbm, cnt_hbm, lpos_hbm, x_hbm,
          xs_hbm, tts_hbm, be_hbm, nb_hbm,
          cv, ts_v, be_v, nb_v, xb0, xb1, o_v, lp_v, off_v, sem0, sem1):
        w = _wid()
        pltpu.sync_copy(cnt_hbm, cv)
        pltpu.sync_copy(o_hbm.at[pl.ds(w * TPW, TPW)], o_v)
        pltpu.sync_copy(lpos_hbm.at[w], lp_v)

        def tsum(t, acc):
            return acc + cv[t]
        tot = lax.fori_loop(0, NW, tsum, jnp.zeros((E,), jnp.int32))
        nblk = (tot + (BLK - 1)) // BLK
        sizes = nblk * BLK
        base = plsc.cumsum(sizes) - sizes

        def osum(t, acc):
            return acc + cv[t]
        off = base + lax.fori_loop(0, w, osum, jnp.zeros((E,), jnp.int32))
        off_v[...] = off

        for kk in range(TPW // L):
            ov = o_v[pl.ds(kk * L, L)]
            lp = lp_v[pl.ds(kk * L, L)]
            ts = plsc.load_gather(off_v, [ov]) + lp
            ts_v[kk // 2, pl.ds((kk % 2) * L, L)] = ts
        pltpu.sync_copy(ts_v, tts_hbm.at[w])

        @pl.when(w == 0)
        def _():
            nb_used = jnp.sum(nblk)
            for g in range(NBMAX // L):
                starts = (lax.iota(jnp.int32, L) + g * L) * BLK
                acc = jnp.zeros((L,), jnp.int32)
                for e in range(E):
                    acc = acc + (base[e] <= starts).astype(jnp.int32)
                be_v[pl.ds(g * L, L)] = acc - 1
            nb_v[...] = jnp.full((L,), nb_used, jnp.int32)
            pltpu.sync_copy(be_v, be_hbm)
            pltpu.sync_copy(nb_v, nb_hbm)

        pending = [None, None]
        for j in range(NCH):
            xb = xb0 if j % 2 == 0 else xb1
            sem = sem0 if j % 2 == 0 else sem1
            if pending[j % 2] is not None:
                pending[j % 2].wait()
            pltpu.sync_copy(x_hbm.at[pl.ds(w * TPW + j * XCH, XCH)], xb)
            cp = pltpu.make_async_copy(xb, xs_hbm.at[ts_v.at[j]], sem)
            cp.start()
            pending[j % 2] = cp
        pending[0].wait()
        pending[1].wait()

    return k(o, counts, lpos, x)


def _tc_mlp(be, nbv, xs, W1, b1r, W2, b2r, Wm, bmr, Ws, bsr,
            NBMAX, D, H, A):
    def body(be_ref, nb_ref, xs_ref, W1_ref, b1_ref, W2_ref, b2_ref,
             Wm_ref, bm_ref, Ws_ref, bs_ref, out_ref):
        i = pl.program_id(0)

        @pl.when(i < nb_ref[0])
        def _():
            hi = lax.Precision.DEFAULT
            xb = xs_ref[...]
            h1 = jnp.maximum(
                jnp.dot(xb, W1_ref[0], precision=hi,
                        preferred_element_type=jnp.float32) + b1_ref[0, 0], 0.0)
            h2 = jnp.maximum(
                jnp.dot(h1, W2_ref[0], precision=hi,
                        preferred_element_type=jnp.float32) + b2_ref[0, 0], 0.0)
            mean = jnp.dot(h2, Wm_ref[0], precision=hi,
                           preferred_element_type=jnp.float32) + bm_ref[0, 0]
            ls = jnp.dot(h2, Ws_ref[0], precision=hi,
                         preferred_element_type=jnp.float32) + bs_ref[0, 0]
            ls = LS_MIN + 0.5 * (LS_MAX - LS_MIN) * (jnp.tanh(ls) + 1.0)
            out_ref[...] = jnp.concatenate([mean, ls], axis=1)

    def xmap(i, be_r, nb_r):
        return (jnp.where(i < nb_r[0], i, nb_r[0] - 1), 0)

    def emap(i, be_r, nb_r):
        return (be_r[i], 0, 0)

    grid_spec = pltpu.PrefetchScalarGridSpec(
        num_scalar_prefetch=2,
        grid=(NBMAX,),
        in_specs=[
            pl.BlockSpec((BLK, D), xmap),
            pl.BlockSpec((1, D, H), emap),
            pl.BlockSpec((1, 1, H), emap),
            pl.BlockSpec((1, H, H), emap),
            pl.BlockSpec((1, 1, H), emap),
            pl.BlockSpec((1, H, A), emap),
            pl.BlockSpec((1, 1, A), emap),
            pl.BlockSpec((1, H, A), emap),
            pl.BlockSpec((1, 1, A), emap),
        ],
        out_specs=pl.BlockSpec((BLK, 2 * A), lambda i, be_r, nb_r: (i, 0)),
    )
    return pl.pallas_call(
        body,
        grid_spec=grid_spec,
        out_shape=jax.ShapeDtypeStruct((NBMAX * BLK, 2 * A), jnp.float32),
    )(be, nbv, xs, W1, b1r, W2, b2r, Wm, bmr, Ws, bsr)


def _sc_unsort(tts, comb, B, W2A):
    TPW = B // NW
    NCH = TPW // XCH

    @functools.partial(
        pl.kernel, mesh=_mesh(),
        compiler_params=pltpu.CompilerParams(needs_layout_passes=False),
        out_type=jax.ShapeDtypeStruct((B, W2A), jnp.float32),
        scratch_types=[
            pltpu.VMEM((NCH, XCH), jnp.int32),
            pltpu.VMEM((XCH, W2A), jnp.float32),
            pltpu.VMEM((XCH, W2A), jnp.float32),
            pltpu.SemaphoreType.DMA,
            pltpu.SemaphoreType.DMA,
        ],
    )
    def k(tts_hbm, comb_hbm, out_hbm, ts_v, g0, g1, sem0, sem1):
        w = _wid()
        pltpu.sync_copy(tts_hbm.at[w], ts_v)
        bufs = (g0, g1)
        sems = (sem0, sem1)
        pending = [None, None]
        pending[0] = pltpu.make_async_copy(comb_hbm.at[ts_v.at[0]],
                                           bufs[0], sems[0])
        pending[0].start()
        for j in range(NCH):
            if j + 1 < NCH:
                nxt = pltpu.make_async_copy(comb_hbm.at[ts_v.at[j + 1]],
                                            bufs[(j + 1) % 2], sems[(j + 1) % 2])
                nxt.start()
                pending[(j + 1) % 2] = nxt
            pending[j % 2].wait()
            pltpu.sync_copy(bufs[j % 2],
                            out_hbm.at[pl.ds(w * TPW + j * XCH, XCH)])

    return k(tts, comb)


def kernel(x, o, W1, b1, W2, b2, Wm, bm, Ws, bs):
    B, D = x.shape
    E, _, H = W1.shape
    A = Wm.shape[2]
    NBMAX = B // BLK + E

    o32 = o.astype(jnp.int32)
    counts, lpos = _sc_count(o32, B, E)
    xs, tts, be, nbv = _sc_dispatch(o32, counts, lpos, x, B, D, E, NBMAX)
    comb = _tc_mlp(be, nbv, xs, W1, b1[:, None, :], W2, b2[:, None, :],
                   Wm, bm[:, None, :], Ws, bs[:, None, :], NBMAX, D, H, A)
    out = _sc_unsort(tts, comb, B, 2 * A)
    return out[:, :A], out[:, A:]


# speedup vs baseline: 2.7181x; 1.0052x over previous
import functools

import jax
import jax.numpy as jnp
from jax import lax
from jax.experimental import pallas as pl
from jax.experimental.pallas import tpu as pltpu
from jax.experimental.pallas import tpu_sc as plsc

LS_MAX = 2.0
LS_MIN = -5.0

NC, NS = 2, 16
NW = NC * NS
L = 16
BLK = 512
XCH = 32


def _wid():
    return lax.axis_index("s") * NC + lax.axis_index("c")


def _mesh():
    return plsc.VectorSubcoreMesh(
        core_axis_name="c", subcore_axis_name="s",
        num_cores=NC, num_subcores=NS)


def _sc_count(o, B, E):
    TPW = B // NW

    @functools.partial(
        pl.kernel, mesh=_mesh(),
        compiler_params=pltpu.CompilerParams(needs_layout_passes=False),
        out_type=(jax.ShapeDtypeStruct((NW, E), jnp.int32),
                  jax.ShapeDtypeStruct((NW, TPW), jnp.int32)),
        scratch_types=[
            pltpu.VMEM((TPW,), jnp.int32),
            pltpu.VMEM((TPW,), jnp.int32),
            pltpu.VMEM((E,), jnp.int32),
        ],
    )
    def k(o_hbm, cnt_hbm, lpos_hbm, o_v, lp_v, c_v):
        w = _wid()
        pltpu.sync_copy(o_hbm.at[pl.ds(w * TPW, TPW)], o_v)
        lanes = lax.iota(jnp.int32, L)

        def sub(kk, cnt):
            ov = o_v[pl.ds(kk * L, L)]
            lp = jnp.zeros((L,), jnp.int32)
            for e in range(E):
                m = ov == e
                m32 = m.astype(jnp.int32)
                csum = plsc.cumsum(m32)
                lp = jnp.where(m, cnt[e] + csum - 1, lp)
                pc = plsc.all_reduce_population_count(m)
                cnt = cnt + jnp.where(lanes == e, pc,
                                      jnp.zeros((L,), jnp.int32))
            lp_v[pl.ds(kk * L, L)] = lp
            return cnt

        cnt = lax.fori_loop(0, TPW // L, sub, jnp.zeros((E,), jnp.int32))
        c_v[...] = cnt
        pltpu.sync_copy(lp_v, lpos_hbm.at[w])
        pltpu.sync_copy(c_v, cnt_hbm.at[w])

    return k(o)


def _sc_dispatch(o, counts, lpos, x, B, D, E, NBMAX):
    TPW = B // NW
    NCH = TPW // XCH
    B_pad = NBMAX * BLK

    @functools.partial(
        pl.kernel, mesh=_mesh(),
        compiler_params=pltpu.CompilerParams(needs_layout_passes=False),
        out_type=(jax.ShapeDtypeStruct((B_pad, D), jnp.float32),
                  jax.ShapeDtypeStruct((NW, NCH, XCH), jnp.int32),
                  jax.ShapeDtypeStruct((NBMAX,), jnp.int32),
                  jax.ShapeDtypeStruct((L,), jnp.int32)),
        scratch_types=[
            pltpu.VMEM((NW, E), jnp.int32),
            pltpu.VMEM((NCH, XCH), jnp.int32),
            pltpu.VMEM((NBMAX,), jnp.int32),
            pltpu.VMEM((L,), jnp.int32),
            pltpu.VMEM((XCH, D), jnp.float32),
            pltpu.VMEM((XCH, D), jnp.float32),
            pltpu.VMEM((TPW,), jnp.int32),
            pltpu.VMEM((TPW,), jnp.int32),
            pltpu.VMEM((E,), jnp.int32),
            pltpu.SemaphoreType.DMA,
            pltpu.SemaphoreType.DMA,
            pltpu.SemaphoreType.DMA,
            pltpu.SemaphoreType.DMA,
        ],
    )
    def k(o_hbm, cnt_hbm, lpos_hbm, x_hbm,
          xs_hbm, tts_hbm, be_hbm, nb_hbm,
          cv, ts_v, be_v, nb_v, xb0, xb1, o_v, lp_v, off_v,
          sem0, sem1, lsem0, lsem1):
        w = _wid()
        pltpu.sync_copy(cnt_hbm, cv)
        pltpu.sync_copy(o_hbm.at[pl.ds(w * TPW, TPW)], o_v)
        pltpu.sync_copy(lpos_hbm.at[w], lp_v)

        def tsum(t, acc):
            return acc + cv[t]
        tot = lax.fori_loop(0, NW, tsum, jnp.zeros((E,), jnp.int32))
        nblk = (tot + (BLK - 1)) // BLK
        sizes = nblk * BLK
        base = plsc.cumsum(sizes) - sizes

        def osum(t, acc):
            return acc + cv[t]
        off = base + lax.fori_loop(0, w, osum, jnp.zeros((E,), jnp.int32))
        off_v[...] = off

        for kk in range(TPW // L):
            ov = o_v[pl.ds(kk * L, L)]
            lp = lp_v[pl.ds(kk * L, L)]
            ts = plsc.load_gather(off_v, [ov]) + lp
            ts_v[kk // 2, pl.ds((kk % 2) * L, L)] = ts
        pltpu.sync_copy(ts_v, tts_hbm.at[w])

        @pl.when(w == 0)
        def _():
            nb_used = jnp.sum(nblk)
            for g in range(NBMAX // L):
                starts = (lax.iota(jnp.int32, L) + g * L) * BLK
                acc = jnp.zeros((L,), jnp.int32)
                for e in range(E):
                    acc = acc + (base[e] <= starts).astype(jnp.int32)
                be_v[pl.ds(g * L, L)] = acc - 1
            nb_v[...] = jnp.full((L,), nb_used, jnp.int32)
            pltpu.sync_copy(be_v, be_hbm)
            pltpu.sync_copy(nb_v, nb_hbm)

        bufs = (xb0, xb1)
        lsems = (lsem0, lsem1)
        ssems = (sem0, sem1)

        def load(j, buf, sem):
            cp = pltpu.make_async_copy(
                x_hbm.at[pl.ds(w * TPW + j * XCH, XCH)], buf, sem)
            cp.start()
            return cp

        pend_load = [load(0, bufs[0], lsems[0]), None]
        pend_scat = [None, None]
        for j in range(NCH):
            p = j % 2
            q = (j + 1) % 2
            if j + 1 < NCH:
                if pend_scat[q] is not None:
                    pend_scat[q].wait()
                pend_load[q] = load(j + 1, bufs[q], lsems[q])
            pend_load[p].wait()
            cp = pltpu.make_async_copy(bufs[p], xs_hbm.at[ts_v.at[j]],
                                       ssems[p])
            cp.start()
            pend_scat[p] = cp
        pend_scat[0].wait()
        pend_scat[1].wait()

    return k(o, counts, lpos, x)


def _tc_mlp(be, nbv, xs, W1, b1r, W2, b2r, Wm, bmr, Ws, bsr,
            NBMAX, D, H, A):
    def body(be_ref, nb_ref, xs_ref, W1_ref, b1_ref, W2_ref, b2_ref,
             Wm_ref, bm_ref, Ws_ref, bs_ref, out_ref):
        i = pl.program_id(0)

        @pl.when(i < nb_ref[0])
        def _():
            hi = lax.Precision.DEFAULT
            xb = xs_ref[...]
            h1 = jnp.maximum(
                jnp.dot(xb, W1_ref[0], precision=hi,
                        preferred_element_type=jnp.float32) + b1_ref[0, 0], 0.0)
            h2 = jnp.maximum(
                jnp.dot(h1, W2_ref[0], precision=hi,
                        preferred_element_type=jnp.float32) + b2_ref[0, 0], 0.0)
            mean = jnp.dot(h2, Wm_ref[0], precision=hi,
                           preferred_element_type=jnp.float32) + bm_ref[0, 0]
            ls = jnp.dot(h2, Ws_ref[0], precision=hi,
                         preferred_element_type=jnp.float32) + bs_ref[0, 0]
            ls = LS_MIN + 0.5 * (LS_MAX - LS_MIN) * (jnp.tanh(ls) + 1.0)
            out_ref[...] = jnp.concatenate([mean, ls], axis=1)

    def xmap(i, be_r, nb_r):
        return (jnp.where(i < nb_r[0], i, nb_r[0] - 1), 0)

    def emap(i, be_r, nb_r):
        return (be_r[i], 0, 0)

    grid_spec = pltpu.PrefetchScalarGridSpec(
        num_scalar_prefetch=2,
        grid=(NBMAX,),
        in_specs=[
            pl.BlockSpec((BLK, D), xmap),
            pl.BlockSpec((1, D, H), emap),
            pl.BlockSpec((1, 1, H), emap),
            pl.BlockSpec((1, H, H), emap),
            pl.BlockSpec((1, 1, H), emap),
            pl.BlockSpec((1, H, A), emap),
            pl.BlockSpec((1, 1, A), emap),
            pl.BlockSpec((1, H, A), emap),
            pl.BlockSpec((1, 1, A), emap),
        ],
        out_specs=pl.BlockSpec((BLK, 2 * A), lambda i, be_r, nb_r: (i, 0)),
    )
    return pl.pallas_call(
        body,
        grid_spec=grid_spec,
        out_shape=jax.ShapeDtypeStruct((NBMAX * BLK, 2 * A), jnp.float32),
    )(be, nbv, xs, W1, b1r, W2, b2r, Wm, bmr, Ws, bsr)


def _sc_unsort(tts, comb, B, W2A):
    TPW = B // NW
    NCH = TPW // XCH

    @functools.partial(
        pl.kernel, mesh=_mesh(),
        compiler_params=pltpu.CompilerParams(needs_layout_passes=False),
        out_type=jax.ShapeDtypeStruct((B, W2A), jnp.float32),
        scratch_types=[
            pltpu.VMEM((NCH, XCH), jnp.int32),
            pltpu.VMEM((XCH, W2A), jnp.float32),
            pltpu.VMEM((XCH, W2A), jnp.float32),
            pltpu.SemaphoreType.DMA,
            pltpu.SemaphoreType.DMA,
        ],
    )
    def k(tts_hbm, comb_hbm, out_hbm, ts_v, g0, g1, sem0, sem1):
        w = _wid()
        pltpu.sync_copy(tts_hbm.at[w], ts_v)
        bufs = (g0, g1)
        sems = (sem0, sem1)
        pending = [None, None]
        pending[0] = pltpu.make_async_copy(comb_hbm.at[ts_v.at[0]],
                                           bufs[0], sems[0])
        pending[0].start()
        for j in range(NCH):
            if j + 1 < NCH:
                nxt = pltpu.make_async_copy(comb_hbm.at[ts_v.at[j + 1]],
                                            bufs[(j + 1) % 2], sems[(j + 1) % 2])
                nxt.start()
                pending[(j + 1) % 2] = nxt
            pending[j % 2].wait()
            pltpu.sync_copy(bufs[j % 2],
                            out_hbm.at[pl.ds(w * TPW + j * XCH, XCH)])

    return k(tts, comb)


def kernel(x, o, W1, b1, W2, b2, Wm, bm, Ws, bs):
    B, D = x.shape
    E, _, H = W1.shape
    A = Wm.shape[2]
    NBMAX = B // BLK + E

    o32 = o.astype(jnp.int32)
    counts, lpos = _sc_count(o32, B, E)
    xs, tts, be, nbv = _sc_dispatch(o32, counts, lpos, x, B, D, E, NBMAX)
    comb = _tc_mlp(be, nbv, xs, W1, b1[:, None, :], W2, b2[:, None, :],
                   Wm, bm[:, None, :], Ws, bs[:, None, :], NBMAX, D, H, A)
    out = _sc_unsort(tts, comb, B, 2 * A)
    return out[:, :A], out[:, A:]
